# baseline (speedup 1.0000x reference)
import functools

import jax
import jax.numpy as jnp
from jax import lax
from jax.experimental import pallas as pl
from jax.experimental.pallas import tpu as pltpu
from jax.experimental.pallas import tpu_sc as plsc

NC = 2
NS = 16
LANES = 16


def _make_kernel(B, F, V, E):
    fd = V // F
    seg_len = ((fd + 127) // 128 + 1) * 128
    mesh = plsc.VectorSubcoreMesh(core_axis_name="c", subcore_axis_name="s")

    @functools.partial(
        pl.kernel,
        out_type=jax.ShapeDtypeStruct((F, E, B), jnp.float32),
        mesh=mesh,
        scratch_types=[
            pltpu.VMEM((seg_len,), jnp.float32),
            pltpu.VMEM((seg_len,), jnp.float32),
            pltpu.VMEM((B,), jnp.int32),
            pltpu.VMEM((B,), jnp.float32),
            pltpu.VMEM((B,), jnp.float32),
            pltpu.SemaphoreType.DMA,
            pltpu.SemaphoreType.DMA,
            pltpu.SemaphoreType.DMA,
            pltpu.SemaphoreType.DMA,
        ],
        compiler_params=pltpu.CompilerParams(
            needs_layout_passes=False, skip_device_barrier=True
        ),
    )
    def k(xt_hbm, tabt_hbm, out_hbm, seg_a, seg_b, idx, row_a, row_b,
          sg0, sg1, sr0, sr1):
        wid = lax.axis_index("s") * NC + lax.axis_index("c")

        @pl.when(wid < F)
        def _():
            f = wid
            seg0 = f * fd
            start = seg0 // 128 * 128
            delta = seg0 - start
            sgs = (sg0, sg1)
            srs = (sr0, sr1)
            segs = (seg_a, seg_b)
            rows = (row_a, row_b)
            cp_seg = [None] * E
            cp_out = [None] * E
            cp_seg[0] = pltpu.async_copy(
                tabt_hbm.at[0, pl.ds(start, seg_len)], segs[0], sgs[0]
            )
            pltpu.sync_copy(xt_hbm.at[f], idx)
            for e in range(E):
                b = e & 1
                if e + 1 < E:
                    cp_seg[e + 1] = pltpu.async_copy(
                        tabt_hbm.at[e + 1, pl.ds(start, seg_len)],
                        segs[1 - b],
                        sgs[1 - b],
                    )
                cp_seg[e].wait()
                if e >= 2:
                    cp_out[e - 2].wait()

                def gather_body(i, b=b):
                    iv = idx[pl.ds(i * LANES, LANES)] + delta
                    rows[b][pl.ds(i * LANES, LANES)] = plsc.load_gather(
                        segs[b], [iv]
                    )

                plsc.parallel_loop(0, B // LANES, unroll=8)(gather_body)
                cp_out[e] = pltpu.async_copy(rows[b], out_hbm.at[f, e], srs[b])
            cp_out[E - 2].wait()
            cp_out[E - 1].wait()

    return k


def kernel(x, table):
    B, F = x.shape
    V, E = table.shape
    out = _make_kernel(B, F, V, E)(x.T.astype(jnp.int32), table.T)
    return out.transpose(2, 0, 1)


# speedup vs baseline: 1.0432x; 1.0432x over previous
import functools

import jax
import jax.numpy as jnp
from jax import lax
from jax.experimental import pallas as pl
from jax.experimental.pallas import tpu as pltpu
from jax.experimental.pallas import tpu_sc as plsc

NC = 2
NS = 16
LANES = 16


def _make_kernel(B, F, V, E):
    fd = V // F
    seg_len = ((fd + 127) // 128 + 1) * 128
    per_w = F * E // (NC * NS)
    mesh = plsc.VectorSubcoreMesh(core_axis_name="c", subcore_axis_name="s")

    @functools.partial(
        pl.kernel,
        out_type=jax.ShapeDtypeStruct((F, E, B), jnp.float32),
        mesh=mesh,
        scratch_types=[
            pltpu.VMEM((seg_len,), jnp.float32),
            pltpu.VMEM((seg_len,), jnp.float32),
            pltpu.VMEM((B,), jnp.int32),
            pltpu.VMEM((B,), jnp.float32),
            pltpu.VMEM((B,), jnp.float32),
            pltpu.SemaphoreType.DMA,
            pltpu.SemaphoreType.DMA,
            pltpu.SemaphoreType.DMA,
            pltpu.SemaphoreType.DMA,
        ],
        compiler_params=pltpu.CompilerParams(needs_layout_passes=False),
    )
    def k(xt_hbm, tabt_hbm, out_hbm, seg_a, seg_b, idx, row_a, row_b,
          sg0, sg1, sr0, sr1):
        wid = lax.axis_index("s") * NC + lax.axis_index("c")
        base = wid * per_w

        def params(j):
            p = base + j
            f = p // E
            e = p % E
            seg0 = f * fd
            start = seg0 // 128 * 128
            return f, e, start, seg0 - start

        sgs = (sg0, sg1)
        srs = (sr0, sr1)
        segs = (seg_a, seg_b)
        rows = (row_a, row_b)
        cp_seg = [None] * per_w
        cp_out = [None] * per_w
        f0, e0, st0, _ = params(0)
        cp_seg[0] = pltpu.async_copy(
            tabt_hbm.at[e0, pl.ds(st0, seg_len)], segs[0], sgs[0]
        )
        pltpu.sync_copy(xt_hbm.at[f0], idx)
        for j in range(per_w):
            fj, ej, stj, dj = params(j)
            b = j & 1
            if j + 1 < per_w:
                fn, en, stn, _ = params(j + 1)
                cp_seg[j + 1] = pltpu.async_copy(
                    tabt_hbm.at[en, pl.ds(stn, seg_len)],
                    segs[1 - b],
                    sgs[1 - b],
                )
            if j > 0:
                fjm = (base + j - 1) // E

                @pl.when(fj != fjm)
                def _(fj=fj):
                    pltpu.sync_copy(xt_hbm.at[fj], idx)

            cp_seg[j].wait()
            if j >= 2:
                cp_out[j - 2].wait()

            def gather_body(i, b=b, dj=dj):
                iv = idx[pl.ds(i * LANES, LANES)] + dj
                rows[b][pl.ds(i * LANES, LANES)] = plsc.load_gather(
                    segs[b], [iv]
                )

            plsc.parallel_loop(0, B // LANES, unroll=8)(gather_body)
            cp_out[j] = pltpu.async_copy(rows[b], out_hbm.at[fj, ej], srs[b])
        cp_out[per_w - 2].wait()
        cp_out[per_w - 1].wait()

    return k


def kernel(x, table):
    B, F = x.shape
    V, E = table.shape
    out = _make_kernel(B, F, V, E)(x.T.astype(jnp.int32), table.T)
    return out.transpose(2, 0, 1)
